# baseline (speedup 1.0000x reference)
import functools

import jax
import jax.numpy as jnp
from jax import lax
from jax.experimental import pallas as pl
from jax.experimental.pallas import tpu as pltpu
from jax.experimental.pallas import tpu_sc as plsc

NUM_EXPERTS = 8
TOP_K = 2
NUM_SHARED = 2
DIM = 768
HIDDEN = 1024
TOKENS = 2048

BT = 256
BTE = 256
NT = TOKENS * TOP_K // BTE + NUM_EXPERTS
ROWS = NT * BTE

NUM_WORKERS = 32
TPW = TOKENS // NUM_WORKERS
CHUNK = 32


def _gating_body(x_ref, wg_ref, pos0_ref, pos1_ref, w0_ref, w1_ref,
                 te_ref, aux_ref):
    x = x_ref[...]
    wg = wg_ref[...]
    logits = lax.dot_general(x, wg, (((1,), (1,)), ((), ())),
                             preferred_element_type=jnp.float32)
    m = jnp.max(logits, axis=-1, keepdims=True)
    ex = jnp.exp(logits - m)
    probs = ex / jnp.sum(ex, axis=-1, keepdims=True)

    eidx = lax.broadcasted_iota(jnp.int32, (TOKENS, NUM_EXPERTS), 1)
    v0 = jnp.max(probs, axis=-1, keepdims=True)
    i0 = jnp.min(jnp.where(probs == v0, eidx, NUM_EXPERTS), axis=-1,
                 keepdims=True)
    probs2 = jnp.where(eidx == i0, -jnp.inf, probs)
    v1 = jnp.max(probs2, axis=-1, keepdims=True)
    i1 = jnp.min(jnp.where(probs2 == v1, eidx, NUM_EXPERTS), axis=-1,
                 keepdims=True)
    wsum = v0 + v1
    w0_ref[...] = jnp.broadcast_to(v0 / wsum, (TOKENS, 16))
    w1_ref[...] = jnp.broadcast_to(v1 / wsum, (TOKENS, 16))

    density = jnp.mean(probs, axis=0)
    proxy = jnp.mean(logits, axis=0)
    aux_ref[...] = jnp.sum(density * proxy).reshape(1, 1) * NUM_EXPERTS

    h0 = (eidx == i0).astype(jnp.int32)
    h1 = (eidx == i1).astype(jnp.int32)
    nb = TOKENS // 128
    hr = (h0 + h1).astype(jnp.float32).reshape(nb, 128, NUM_EXPERTS)
    ri = lax.broadcasted_iota(jnp.int32, (128, 128), 0)
    ci = lax.broadcasted_iota(jnp.int32, (128, 128), 1)
    tri = jnp.broadcast_to((ri >= ci).astype(jnp.float32),
                           (nb, 128, 128))
    cw = lax.dot_general(tri, hr, (((2,), (1,)), ((0,), (0,))),
                         preferred_element_type=jnp.float32)
    s = cw[:, 127, :]
    sx = s
    shift = 1
    while shift < nb:
        sx = sx + jnp.concatenate(
            [jnp.zeros((shift, NUM_EXPERTS), jnp.float32), sx[:-shift]],
            axis=0)
        shift *= 2
    c = (cw + (sx - s)[:, None, :]).reshape(TOKENS, NUM_EXPERTS)
    c = c.astype(jnp.int32)
    counts = sx[nb - 1:nb, :].astype(jnp.int32)
    ntiles = (counts + (BTE - 1)) // BTE
    ct = ntiles
    shift = 1
    while shift < NUM_EXPERTS:
        ct = ct + jnp.concatenate(
            [jnp.zeros((1, shift), jnp.int32), ct[:, :-shift]], axis=1)
        shift *= 2
    cum = jnp.concatenate([jnp.zeros((1, 1), jnp.int32), ct], axis=1)
    row_off = cum[:, :NUM_EXPERTS] * BTE

    pos_of = lambda h: jnp.sum(h * (row_off + c - 1), axis=1)
    pos0_ref[...] = pos_of(h0)
    pos1_ref[...] = pos_of(h1)

    tidx = lax.broadcasted_iota(jnp.int32, (NT, NUM_EXPERTS), 0)
    te = jnp.sum((tidx >= cum[0, 1:][None, :]).astype(jnp.int32), axis=1)
    te_ref[...] = jnp.concatenate(
        [jnp.minimum(te, NUM_EXPERTS - 1), ct[0, -1:]], axis=0)


def _gating(x, wg):
    return pl.pallas_call(
        _gating_body,
        out_shape=(
            jax.ShapeDtypeStruct((TOKENS,), jnp.int32),
            jax.ShapeDtypeStruct((TOKENS,), jnp.int32),
            jax.ShapeDtypeStruct((TOKENS, 16), jnp.float32),
            jax.ShapeDtypeStruct((TOKENS, 16), jnp.float32),
            jax.ShapeDtypeStruct((NT + 1,), jnp.int32),
            jax.ShapeDtypeStruct((1, 1), jnp.float32),
        ),
    )(x, wg)


def _scatter_body(x_hbm, pos0_hbm, pos1_hbm, xs_hbm, x_v, i0_v, i1_v,
                  sem0, sem1):
    wid = lax.axis_index("s") * 2 + lax.axis_index("c")
    base = wid * TPW
    pltpu.sync_copy(x_hbm.at[pl.ds(base, TPW)], x_v)
    pltpu.sync_copy(pos0_hbm.at[pl.ds(base, TPW)], i0_v)
    pltpu.sync_copy(pos1_hbm.at[pl.ds(base, TPW)], i1_v)
    c0 = pltpu.async_copy(x_v, xs_hbm.at[i0_v], sem0)
    c1 = pltpu.async_copy(x_v, xs_hbm.at[i1_v], sem1)
    c0.wait()
    c1.wait()


@functools.cache
def _sc_scatter():
    return pl.kernel(
        _scatter_body,
        out_type=jax.ShapeDtypeStruct((ROWS, DIM), jnp.float32),
        mesh=plsc.VectorSubcoreMesh(core_axis_name="c",
                                    subcore_axis_name="s"),
        scratch_types=[
            pltpu.VMEM((TPW, DIM), jnp.float32),
            pltpu.VMEM((TPW,), jnp.int32),
            pltpu.VMEM((TPW,), jnp.int32),
            pltpu.SemaphoreType.DMA,
            pltpu.SemaphoreType.DMA,
        ],
    )


def _expert_body(te_ref, x_ref, w1_ref, w2_ref, y_ref):
    @pl.when(pl.program_id(0) < te_ref[NT])
    def _():
        xb = x_ref[...]
        h = lax.dot_general(xb, w1_ref[0], (((1,), (1,)), ((), ())),
                            preferred_element_type=jnp.float32)
        h = h * jax.nn.sigmoid(h)
        y_ref[...] = lax.dot_general(h, w2_ref[0], (((1,), (1,)), ((), ())),
                                     preferred_element_type=jnp.float32)


def _expert_gemm(te, xs, wr1, wr2):
    grid_spec = pltpu.PrefetchScalarGridSpec(
        num_scalar_prefetch=1,
        grid=(NT,),
        in_specs=[
            pl.BlockSpec((BTE, DIM), lambda i, te: (i, 0)),
            pl.BlockSpec((1, HIDDEN, DIM), lambda i, te: (te[i], 0, 0)),
            pl.BlockSpec((1, DIM, HIDDEN), lambda i, te: (te[i], 0, 0)),
        ],
        out_specs=pl.BlockSpec((BTE, DIM), lambda i, te: (i, 0)),
    )
    return pl.pallas_call(
        _expert_body,
        grid_spec=grid_spec,
        out_shape=jax.ShapeDtypeStruct((ROWS, DIM), jnp.float32),
    )(te, xs, wr1, wr2)


def _shared_body(x_ref, w1_ref, w2_ref, o_ref):
    xb = x_ref[...]
    acc = jnp.zeros((BT, DIM), jnp.float32)
    for s in range(NUM_SHARED):
        h = lax.dot_general(xb, w1_ref[s], (((1,), (1,)), ((), ())),
                            preferred_element_type=jnp.float32)
        h = h * jax.nn.sigmoid(h)
        acc = acc + lax.dot_general(h, w2_ref[s], (((1,), (1,)), ((), ())),
                                    preferred_element_type=jnp.float32)
    o_ref[...] = acc


def _shared(x, ws1, ws2):
    n = x.shape[0]
    return pl.pallas_call(
        _shared_body,
        grid=(n // BT,),
        in_specs=[
            pl.BlockSpec((BT, DIM), lambda i: (i, 0)),
            pl.BlockSpec((NUM_SHARED, HIDDEN, DIM), lambda i: (0, 0, 0)),
            pl.BlockSpec((NUM_SHARED, DIM, HIDDEN), lambda i: (0, 0, 0)),
        ],
        out_specs=pl.BlockSpec((BT, DIM), lambda i: (i, 0)),
        out_shape=jax.ShapeDtypeStruct((n, DIM), jnp.float32),
    )(x, ws1, ws2)


def _gather_body(y_hbm, pos0_hbm, pos1_hbm, o0_hbm, o1_hbm,
                 y0_v, y1_v, i0_v, i1_v, sem0, sem1):
    wid = lax.axis_index("s") * 2 + lax.axis_index("c")
    base = wid * TPW
    pltpu.sync_copy(pos0_hbm.at[pl.ds(base, TPW)], i0_v)
    pltpu.sync_copy(pos1_hbm.at[pl.ds(base, TPW)], i1_v)
    c0 = pltpu.async_copy(y_hbm.at[i0_v], y0_v, sem0)
    c1 = pltpu.async_copy(y_hbm.at[i1_v], y1_v, sem1)
    c0.wait()
    c1.wait()
    pltpu.sync_copy(y0_v, o0_hbm.at[pl.ds(base, TPW)])
    pltpu.sync_copy(y1_v, o1_hbm.at[pl.ds(base, TPW)])


@functools.cache
def _sc_gather():
    return pl.kernel(
        _gather_body,
        out_type=(
            jax.ShapeDtypeStruct((TOKENS, DIM), jnp.float32),
            jax.ShapeDtypeStruct((TOKENS, DIM), jnp.float32),
        ),
        mesh=plsc.VectorSubcoreMesh(core_axis_name="c",
                                    subcore_axis_name="s"),
        scratch_types=[
            pltpu.VMEM((TPW, DIM), jnp.float32),
            pltpu.VMEM((TPW, DIM), jnp.float32),
            pltpu.VMEM((TPW,), jnp.int32),
            pltpu.VMEM((TPW,), jnp.int32),
            pltpu.SemaphoreType.DMA,
            pltpu.SemaphoreType.DMA,
        ],
    )


def _combine_body(sh_ref, y0_ref, y1_ref, w0_ref, w1_ref, o_ref):
    w0 = w0_ref[:, 0:1]
    w1 = w1_ref[:, 0:1]
    o_ref[...] = sh_ref[...] + w0 * y0_ref[...] + w1 * y1_ref[...]


def _combine(sh, y0g, y1g, w0, w1):
    return pl.pallas_call(
        _combine_body,
        grid=(TOKENS // BT,),
        in_specs=[
            pl.BlockSpec((BT, DIM), lambda i: (i, 0)),
            pl.BlockSpec((BT, DIM), lambda i: (i, 0)),
            pl.BlockSpec((BT, DIM), lambda i: (i, 0)),
            pl.BlockSpec((BT, 16), lambda i: (i, 0)),
            pl.BlockSpec((BT, 16), lambda i: (i, 0)),
        ],
        out_specs=pl.BlockSpec((BT, DIM), lambda i: (i, 0)),
        out_shape=jax.ShapeDtypeStruct((TOKENS, DIM), jnp.float32),
    )(sh, y0g, y1g, w0, w1)


@jax.jit
def kernel(x, Wg, Ws1, Ws2, Wr1, Wr2):
    shape = x.shape
    xf = x.reshape(TOKENS, DIM)
    pos0, pos1, w0, w1, te, aux = _gating(xf, Wg)
    half = TOKENS // 2
    sh_a = _shared(xf[:half], Ws1, Ws2)
    xs = _sc_scatter()(xf, pos0, pos1)
    y = _expert_gemm(te, xs, Wr1, Wr2)
    sh_b = _shared(xf[half:], Ws1, Ws2)
    y0g, y1g = _sc_gather()(y, pos0, pos1)
    sh = jnp.concatenate([sh_a, sh_b], axis=0)
    out = _combine(sh, y0g, y1g, w0, w1)
    return out.reshape(shape), aux[0, 0]


# speedup vs baseline: 1.2074x; 1.2074x over previous
import functools

import jax
import jax.numpy as jnp
from jax import lax
from jax.experimental import pallas as pl
from jax.experimental.pallas import tpu as pltpu
from jax.experimental.pallas import tpu_sc as plsc

NUM_EXPERTS = 8
TOP_K = 2
NUM_SHARED = 2
DIM = 768
HIDDEN = 1024
TOKENS = 2048

BT = 256
BTE = 256
NT = TOKENS * TOP_K // BTE + NUM_EXPERTS
ROWS = NT * BTE

NUM_WORKERS = 32
TPW = TOKENS // NUM_WORKERS
CHUNK = 32


def _gating_body(x_ref, wg_ref, pos0_ref, pos1_ref, w0_ref, w1_ref,
                 te_ref, aux_ref):
    x = x_ref[...]
    wg = wg_ref[...]
    logits = lax.dot_general(x, wg, (((1,), (1,)), ((), ())),
                             preferred_element_type=jnp.float32)
    m = jnp.max(logits, axis=-1, keepdims=True)
    ex = jnp.exp(logits - m)
    probs = ex / jnp.sum(ex, axis=-1, keepdims=True)

    eidx = lax.broadcasted_iota(jnp.int32, (TOKENS, NUM_EXPERTS), 1)
    v0 = jnp.max(probs, axis=-1, keepdims=True)
    i0 = jnp.min(jnp.where(probs == v0, eidx, NUM_EXPERTS), axis=-1,
                 keepdims=True)
    probs2 = jnp.where(eidx == i0, -jnp.inf, probs)
    v1 = jnp.max(probs2, axis=-1, keepdims=True)
    i1 = jnp.min(jnp.where(probs2 == v1, eidx, NUM_EXPERTS), axis=-1,
                 keepdims=True)
    wsum = v0 + v1
    w0_ref[...] = jnp.broadcast_to(v0 / wsum, (TOKENS, 16))
    w1_ref[...] = jnp.broadcast_to(v1 / wsum, (TOKENS, 16))

    density = jnp.mean(probs, axis=0)
    proxy = jnp.mean(logits, axis=0)
    aux_ref[...] = jnp.sum(density * proxy).reshape(1, 1) * NUM_EXPERTS

    h0 = (eidx == i0).astype(jnp.int32)
    h1 = (eidx == i1).astype(jnp.int32)
    nb = TOKENS // 128
    hr = (h0 + h1).astype(jnp.float32).reshape(nb, 128, NUM_EXPERTS)
    ri = lax.broadcasted_iota(jnp.int32, (128, 128), 0)
    ci = lax.broadcasted_iota(jnp.int32, (128, 128), 1)
    tri = jnp.broadcast_to((ri >= ci).astype(jnp.float32),
                           (nb, 128, 128))
    cw = lax.dot_general(tri, hr, (((2,), (1,)), ((0,), (0,))),
                         preferred_element_type=jnp.float32)
    s = cw[:, 127, :]
    sx = s
    shift = 1
    while shift < nb:
        sx = sx + jnp.concatenate(
            [jnp.zeros((shift, NUM_EXPERTS), jnp.float32), sx[:-shift]],
            axis=0)
        shift *= 2
    c = (cw + (sx - s)[:, None, :]).reshape(TOKENS, NUM_EXPERTS)
    c = c.astype(jnp.int32)
    counts = sx[nb - 1:nb, :].astype(jnp.int32)
    ntiles = (counts + (BTE - 1)) // BTE
    ct = ntiles
    shift = 1
    while shift < NUM_EXPERTS:
        ct = ct + jnp.concatenate(
            [jnp.zeros((1, shift), jnp.int32), ct[:, :-shift]], axis=1)
        shift *= 2
    cum = jnp.concatenate([jnp.zeros((1, 1), jnp.int32), ct], axis=1)
    row_off = cum[:, :NUM_EXPERTS] * BTE

    pos_of = lambda h: jnp.sum(h * (row_off + c - 1), axis=1)
    pos0_ref[...] = pos_of(h0)
    pos1_ref[...] = pos_of(h1)

    tidx = lax.broadcasted_iota(jnp.int32, (NT, NUM_EXPERTS), 0)
    te = jnp.sum((tidx >= cum[0, 1:][None, :]).astype(jnp.int32), axis=1)
    te_ref[...] = jnp.concatenate(
        [jnp.minimum(te, NUM_EXPERTS - 1), ct[0, -1:]], axis=0)


def _gating(x, wg):
    return pl.pallas_call(
        _gating_body,
        out_shape=(
            jax.ShapeDtypeStruct((TOKENS,), jnp.int32),
            jax.ShapeDtypeStruct((TOKENS,), jnp.int32),
            jax.ShapeDtypeStruct((TOKENS, 16), jnp.float32),
            jax.ShapeDtypeStruct((TOKENS, 16), jnp.float32),
            jax.ShapeDtypeStruct((NT + 1,), jnp.int32),
            jax.ShapeDtypeStruct((1, 1), jnp.float32),
        ),
    )(x, wg)


def _scatter_body(x_hbm, pos0_hbm, pos1_hbm, xs_hbm, x_v, i0_v, i1_v,
                  sem0, sem1):
    wid = lax.axis_index("s") * 2 + lax.axis_index("c")
    base = wid * TPW
    pltpu.sync_copy(x_hbm.at[pl.ds(base, TPW)], x_v)
    pltpu.sync_copy(pos0_hbm.at[pl.ds(base, TPW)], i0_v)
    pltpu.sync_copy(pos1_hbm.at[pl.ds(base, TPW)], i1_v)
    c0 = pltpu.async_copy(x_v, xs_hbm.at[i0_v], sem0)
    c1 = pltpu.async_copy(x_v, xs_hbm.at[i1_v], sem1)
    c0.wait()
    c1.wait()


@functools.cache
def _sc_scatter():
    return pl.kernel(
        _scatter_body,
        out_type=jax.ShapeDtypeStruct((ROWS, DIM), jnp.float32),
        mesh=plsc.VectorSubcoreMesh(core_axis_name="c",
                                    subcore_axis_name="s"),
        scratch_types=[
            pltpu.VMEM((TPW, DIM), jnp.float32),
            pltpu.VMEM((TPW,), jnp.int32),
            pltpu.VMEM((TPW,), jnp.int32),
            pltpu.SemaphoreType.DMA,
            pltpu.SemaphoreType.DMA,
        ],
    )


def _expert_body(te_ref, x_ref, w1_ref, w2_ref, y_ref):
    @pl.when(pl.program_id(0) < te_ref[NT])
    def _():
        xb = x_ref[...]
        h = lax.dot_general(xb, w1_ref[0], (((1,), (1,)), ((), ())),
                            preferred_element_type=jnp.float32)
        h = h * jax.nn.sigmoid(h)
        y_ref[...] = lax.dot_general(h, w2_ref[0], (((1,), (1,)), ((), ())),
                                     preferred_element_type=jnp.float32)


def _expert_gemm(te, xs, wr1, wr2):
    grid_spec = pltpu.PrefetchScalarGridSpec(
        num_scalar_prefetch=1,
        grid=(NT,),
        in_specs=[
            pl.BlockSpec((BTE, DIM), lambda i, te: (i, 0)),
            pl.BlockSpec((1, HIDDEN, DIM), lambda i, te: (te[i], 0, 0)),
            pl.BlockSpec((1, DIM, HIDDEN), lambda i, te: (te[i], 0, 0)),
        ],
        out_specs=pl.BlockSpec((BTE, DIM), lambda i, te: (i, 0)),
    )
    return pl.pallas_call(
        _expert_body,
        grid_spec=grid_spec,
        out_shape=jax.ShapeDtypeStruct((ROWS, DIM), jnp.float32),
    )(te, xs, wr1, wr2)


def _shared_body(x_ref, w1_ref, w2_ref, o_ref):
    xb = x_ref[...]
    acc = jnp.zeros((BT, DIM), jnp.float32)
    for s in range(NUM_SHARED):
        h = lax.dot_general(xb, w1_ref[s], (((1,), (1,)), ((), ())),
                            preferred_element_type=jnp.float32)
        h = h * jax.nn.sigmoid(h)
        acc = acc + lax.dot_general(h, w2_ref[s], (((1,), (1,)), ((), ())),
                                    preferred_element_type=jnp.float32)
    o_ref[...] = acc


def _shared(x, ws1, ws2):
    n = x.shape[0]
    return pl.pallas_call(
        _shared_body,
        grid=(n // BT,),
        in_specs=[
            pl.BlockSpec((BT, DIM), lambda i: (i, 0)),
            pl.BlockSpec((NUM_SHARED, HIDDEN, DIM), lambda i: (0, 0, 0)),
            pl.BlockSpec((NUM_SHARED, DIM, HIDDEN), lambda i: (0, 0, 0)),
        ],
        out_specs=pl.BlockSpec((BT, DIM), lambda i: (i, 0)),
        out_shape=jax.ShapeDtypeStruct((n, DIM), jnp.float32),
    )(x, ws1, ws2)


def _gather_body(y_hbm, pos0_hbm, pos1_hbm, w0_hbm, w1_hbm, o_hbm,
                 y0_v, y1_v, i0_v, i1_v, w0_v, w1_v, sem0, sem1):
    wid = lax.axis_index("s") * 2 + lax.axis_index("c")
    base = wid * TPW
    pltpu.sync_copy(pos0_hbm.at[pl.ds(base, TPW)], i0_v)
    pltpu.sync_copy(pos1_hbm.at[pl.ds(base, TPW)], i1_v)
    c0 = pltpu.async_copy(y_hbm.at[i0_v], y0_v, sem0)
    c1 = pltpu.async_copy(y_hbm.at[i1_v], y1_v, sem1)
    pltpu.sync_copy(w0_hbm.at[pl.ds(base, TPW)], w0_v)
    pltpu.sync_copy(w1_hbm.at[pl.ds(base, TPW)], w1_v)
    c0.wait()
    c1.wait()

    def token(i, _):
        a = w0_v[i, :]
        b = w1_v[i, :]
        for j in range(DIM // 16):
            sl = pl.ds(j * 16, 16)
            y0_v[i, sl] = a * y0_v[i, sl] + b * y1_v[i, sl]
        return 0

    lax.fori_loop(0, TPW, token, 0)
    pltpu.sync_copy(y0_v, o_hbm.at[pl.ds(base, TPW)])


@functools.cache
def _sc_gather():
    return pl.kernel(
        _gather_body,
        out_type=jax.ShapeDtypeStruct((TOKENS, DIM), jnp.float32),
        mesh=plsc.VectorSubcoreMesh(core_axis_name="c",
                                    subcore_axis_name="s"),
        scratch_types=[
            pltpu.VMEM((TPW, DIM), jnp.float32),
            pltpu.VMEM((TPW, DIM), jnp.float32),
            pltpu.VMEM((TPW,), jnp.int32),
            pltpu.VMEM((TPW,), jnp.int32),
            pltpu.VMEM((TPW, 16), jnp.float32),
            pltpu.VMEM((TPW, 16), jnp.float32),
            pltpu.SemaphoreType.DMA,
            pltpu.SemaphoreType.DMA,
        ],
    )


def _combine_body(sh_ref, yr_ref, o_ref):
    o_ref[...] = sh_ref[...] + yr_ref[...]


def _combine(sh, yr):
    return pl.pallas_call(
        _combine_body,
        grid=(TOKENS // BT,),
        in_specs=[
            pl.BlockSpec((BT, DIM), lambda i: (i, 0)),
            pl.BlockSpec((BT, DIM), lambda i: (i, 0)),
        ],
        out_specs=pl.BlockSpec((BT, DIM), lambda i: (i, 0)),
        out_shape=jax.ShapeDtypeStruct((TOKENS, DIM), jnp.float32),
    )(sh, yr)


@jax.jit
def kernel(x, Wg, Ws1, Ws2, Wr1, Wr2):
    shape = x.shape
    xf = x.reshape(TOKENS, DIM)
    pos0, pos1, w0, w1, te, aux = _gating(xf, Wg)
    sh = _shared(xf, Ws1, Ws2)
    xs = _sc_scatter()(xf, pos0, pos1)
    y = _expert_gemm(te, xs, Wr1, Wr2)
    yr = _sc_gather()(y, pos0, pos1, w0, w1)
    out = _combine(sh, yr)
    return out.reshape(shape), aux[0, 0]


# speedup vs baseline: 1.2599x; 1.0434x over previous
import functools

import jax
import jax.numpy as jnp
from jax import lax
from jax.experimental import pallas as pl
from jax.experimental.pallas import tpu as pltpu
from jax.experimental.pallas import tpu_sc as plsc

NUM_EXPERTS = 8
TOP_K = 2
NUM_SHARED = 2
DIM = 768
HIDDEN = 1024
TOKENS = 2048

BT = 256
BTE = 512
NT = TOKENS * TOP_K // BTE + NUM_EXPERTS
ROWS = NT * BTE

NUM_WORKERS = 32
TPW = TOKENS // NUM_WORKERS
CHUNK = 32


def _gating_body(x_ref, wg_ref, pos0_ref, pos1_ref, w0_ref, w1_ref,
                 te_ref, aux_ref):
    x = x_ref[...]
    wg = wg_ref[...]
    logits = lax.dot_general(x, wg, (((1,), (1,)), ((), ())),
                             preferred_element_type=jnp.float32)
    m = jnp.max(logits, axis=-1, keepdims=True)
    ex = jnp.exp(logits - m)
    probs = ex / jnp.sum(ex, axis=-1, keepdims=True)

    eidx = lax.broadcasted_iota(jnp.int32, (TOKENS, NUM_EXPERTS), 1)
    v0 = jnp.max(probs, axis=-1, keepdims=True)
    i0 = jnp.min(jnp.where(probs == v0, eidx, NUM_EXPERTS), axis=-1,
                 keepdims=True)
    probs2 = jnp.where(eidx == i0, -jnp.inf, probs)
    v1 = jnp.max(probs2, axis=-1, keepdims=True)
    i1 = jnp.min(jnp.where(probs2 == v1, eidx, NUM_EXPERTS), axis=-1,
                 keepdims=True)
    wsum = v0 + v1
    w0_ref[...] = jnp.broadcast_to(v0 / wsum, (TOKENS, 16))
    w1_ref[...] = jnp.broadcast_to(v1 / wsum, (TOKENS, 16))

    density = jnp.mean(probs, axis=0)
    proxy = jnp.mean(logits, axis=0)
    aux_ref[...] = jnp.sum(density * proxy).reshape(1, 1) * NUM_EXPERTS

    h0 = (eidx == i0).astype(jnp.int32)
    h1 = (eidx == i1).astype(jnp.int32)
    nb = TOKENS // 128
    hr = (h0 + h1).astype(jnp.float32).reshape(nb, 128, NUM_EXPERTS)
    ri = lax.broadcasted_iota(jnp.int32, (128, 128), 0)
    ci = lax.broadcasted_iota(jnp.int32, (128, 128), 1)
    tri = jnp.broadcast_to((ri >= ci).astype(jnp.float32),
                           (nb, 128, 128))
    cw = lax.dot_general(tri, hr, (((2,), (1,)), ((0,), (0,))),
                         preferred_element_type=jnp.float32)
    s = cw[:, 127, :]
    sx = s
    shift = 1
    while shift < nb:
        sx = sx + jnp.concatenate(
            [jnp.zeros((shift, NUM_EXPERTS), jnp.float32), sx[:-shift]],
            axis=0)
        shift *= 2
    c = (cw + (sx - s)[:, None, :]).reshape(TOKENS, NUM_EXPERTS)
    c = c.astype(jnp.int32)
    counts = sx[nb - 1:nb, :].astype(jnp.int32)
    ntiles = (counts + (BTE - 1)) // BTE
    ct = ntiles
    shift = 1
    while shift < NUM_EXPERTS:
        ct = ct + jnp.concatenate(
            [jnp.zeros((1, shift), jnp.int32), ct[:, :-shift]], axis=1)
        shift *= 2
    cum = jnp.concatenate([jnp.zeros((1, 1), jnp.int32), ct], axis=1)
    row_off = cum[:, :NUM_EXPERTS] * BTE

    pos_of = lambda h: jnp.sum(h * (row_off + c - 1), axis=1)
    pos0_ref[...] = pos_of(h0)
    pos1_ref[...] = pos_of(h1)

    tidx = lax.broadcasted_iota(jnp.int32, (NT, NUM_EXPERTS), 0)
    te = jnp.sum((tidx >= cum[0, 1:][None, :]).astype(jnp.int32), axis=1)
    te_ref[...] = jnp.concatenate(
        [jnp.minimum(te, NUM_EXPERTS - 1), ct[0, -1:]], axis=0)


def _gating(x, wg):
    return pl.pallas_call(
        _gating_body,
        out_shape=(
            jax.ShapeDtypeStruct((TOKENS,), jnp.int32),
            jax.ShapeDtypeStruct((TOKENS,), jnp.int32),
            jax.ShapeDtypeStruct((TOKENS, 16), jnp.float32),
            jax.ShapeDtypeStruct((TOKENS, 16), jnp.float32),
            jax.ShapeDtypeStruct((NT + 1,), jnp.int32),
            jax.ShapeDtypeStruct((1, 1), jnp.float32),
        ),
    )(x, wg)


def _scatter_body(x_hbm, pos0_hbm, pos1_hbm, xs_hbm, x_v, i0_v, i1_v,
                  sem0, sem1):
    wid = lax.axis_index("s") * 2 + lax.axis_index("c")
    base = wid * TPW
    pltpu.sync_copy(x_hbm.at[pl.ds(base, TPW)], x_v)
    pltpu.sync_copy(pos0_hbm.at[pl.ds(base, TPW)], i0_v)
    pltpu.sync_copy(pos1_hbm.at[pl.ds(base, TPW)], i1_v)
    c0 = pltpu.async_copy(x_v, xs_hbm.at[i0_v], sem0)
    c1 = pltpu.async_copy(x_v, xs_hbm.at[i1_v], sem1)
    c0.wait()
    c1.wait()


@functools.cache
def _sc_scatter():
    return pl.kernel(
        _scatter_body,
        out_type=jax.ShapeDtypeStruct((ROWS, DIM), jnp.float32),
        mesh=plsc.VectorSubcoreMesh(core_axis_name="c",
                                    subcore_axis_name="s"),
        scratch_types=[
            pltpu.VMEM((TPW, DIM), jnp.float32),
            pltpu.VMEM((TPW,), jnp.int32),
            pltpu.VMEM((TPW,), jnp.int32),
            pltpu.SemaphoreType.DMA,
            pltpu.SemaphoreType.DMA,
        ],
    )


def _expert_body(te_ref, x_ref, w1_ref, w2_ref, y_ref):
    @pl.when(pl.program_id(0) < te_ref[NT])
    def _():
        xb = x_ref[...]
        h = lax.dot_general(xb, w1_ref[0], (((1,), (1,)), ((), ())),
                            preferred_element_type=jnp.float32)
        h = h * jax.nn.sigmoid(h)
        y_ref[...] = lax.dot_general(h, w2_ref[0], (((1,), (1,)), ((), ())),
                                     preferred_element_type=jnp.float32)


def _expert_gemm(te, xs, wr1, wr2):
    grid_spec = pltpu.PrefetchScalarGridSpec(
        num_scalar_prefetch=1,
        grid=(NT,),
        in_specs=[
            pl.BlockSpec((BTE, DIM), lambda i, te: (i, 0)),
            pl.BlockSpec((1, HIDDEN, DIM), lambda i, te: (te[i], 0, 0)),
            pl.BlockSpec((1, DIM, HIDDEN), lambda i, te: (te[i], 0, 0)),
        ],
        out_specs=pl.BlockSpec((BTE, DIM), lambda i, te: (i, 0)),
    )
    return pl.pallas_call(
        _expert_body,
        grid_spec=grid_spec,
        out_shape=jax.ShapeDtypeStruct((ROWS, DIM), jnp.float32),
    )(te, xs, wr1, wr2)


def _shared_body(x_ref, w1_ref, w2_ref, o_ref):
    xb = x_ref[...]
    acc = jnp.zeros((BT, DIM), jnp.float32)
    for s in range(NUM_SHARED):
        h = lax.dot_general(xb, w1_ref[s], (((1,), (1,)), ((), ())),
                            preferred_element_type=jnp.float32)
        h = h * jax.nn.sigmoid(h)
        acc = acc + lax.dot_general(h, w2_ref[s], (((1,), (1,)), ((), ())),
                                    preferred_element_type=jnp.float32)
    o_ref[...] = acc


def _shared(x, ws1, ws2):
    n = x.shape[0]
    return pl.pallas_call(
        _shared_body,
        grid=(n // BT,),
        in_specs=[
            pl.BlockSpec((BT, DIM), lambda i: (i, 0)),
            pl.BlockSpec((NUM_SHARED, HIDDEN, DIM), lambda i: (0, 0, 0)),
            pl.BlockSpec((NUM_SHARED, DIM, HIDDEN), lambda i: (0, 0, 0)),
        ],
        out_specs=pl.BlockSpec((BT, DIM), lambda i: (i, 0)),
        out_shape=jax.ShapeDtypeStruct((n, DIM), jnp.float32),
    )(x, ws1, ws2)


def _gather_body(y_hbm, pos0_hbm, pos1_hbm, w0_hbm, w1_hbm, o_hbm,
                 y0_v, y1_v, i0_v, i1_v, w0_v, w1_v, sem0, sem1):
    wid = lax.axis_index("s") * 2 + lax.axis_index("c")
    base = wid * TPW
    pltpu.sync_copy(pos0_hbm.at[pl.ds(base, TPW)], i0_v)
    pltpu.sync_copy(pos1_hbm.at[pl.ds(base, TPW)], i1_v)
    c0 = pltpu.async_copy(y_hbm.at[i0_v], y0_v, sem0)
    c1 = pltpu.async_copy(y_hbm.at[i1_v], y1_v, sem1)
    pltpu.sync_copy(w0_hbm.at[pl.ds(base, TPW)], w0_v)
    pltpu.sync_copy(w1_hbm.at[pl.ds(base, TPW)], w1_v)
    c0.wait()
    c1.wait()

    def token(i, _):
        a = w0_v[i, :]
        b = w1_v[i, :]
        for j in range(DIM // 16):
            sl = pl.ds(j * 16, 16)
            y0_v[i, sl] = a * y0_v[i, sl] + b * y1_v[i, sl]
        return 0

    lax.fori_loop(0, TPW, token, 0)
    pltpu.sync_copy(y0_v, o_hbm.at[pl.ds(base, TPW)])


@functools.cache
def _sc_gather():
    return pl.kernel(
        _gather_body,
        out_type=jax.ShapeDtypeStruct((TOKENS, DIM), jnp.float32),
        mesh=plsc.VectorSubcoreMesh(core_axis_name="c",
                                    subcore_axis_name="s"),
        scratch_types=[
            pltpu.VMEM((TPW, DIM), jnp.float32),
            pltpu.VMEM((TPW, DIM), jnp.float32),
            pltpu.VMEM((TPW,), jnp.int32),
            pltpu.VMEM((TPW,), jnp.int32),
            pltpu.VMEM((TPW, 16), jnp.float32),
            pltpu.VMEM((TPW, 16), jnp.float32),
            pltpu.SemaphoreType.DMA,
            pltpu.SemaphoreType.DMA,
        ],
    )


def _combine_body(sh_ref, yr_ref, o_ref):
    o_ref[...] = sh_ref[...] + yr_ref[...]


def _combine(sh, yr):
    return pl.pallas_call(
        _combine_body,
        grid=(TOKENS // BT,),
        in_specs=[
            pl.BlockSpec((BT, DIM), lambda i: (i, 0)),
            pl.BlockSpec((BT, DIM), lambda i: (i, 0)),
        ],
        out_specs=pl.BlockSpec((BT, DIM), lambda i: (i, 0)),
        out_shape=jax.ShapeDtypeStruct((TOKENS, DIM), jnp.float32),
    )(sh, yr)


@jax.jit
def kernel(x, Wg, Ws1, Ws2, Wr1, Wr2):
    shape = x.shape
    xf = x.reshape(TOKENS, DIM)
    pos0, pos1, w0, w1, te, aux = _gating(xf, Wg)
    sh = _shared(xf, Ws1, Ws2)
    xs = _sc_scatter()(xf, pos0, pos1)
    y = _expert_gemm(te, xs, Wr1, Wr2)
    yr = _sc_gather()(y, pos0, pos1, w0, w1)
    out = _combine(sh, yr)
    return out.reshape(shape), aux[0, 0]


# speedup vs baseline: 1.3087x; 1.0388x over previous
import functools

import jax
import jax.numpy as jnp
from jax import lax
from jax.experimental import pallas as pl
from jax.experimental.pallas import tpu as pltpu
from jax.experimental.pallas import tpu_sc as plsc

NUM_EXPERTS = 8
TOP_K = 2
NUM_SHARED = 2
DIM = 768
HIDDEN = 1024
TOKENS = 2048

BT = 256
BTE = 768
NT = TOKENS * TOP_K // BTE + NUM_EXPERTS
ROWS = NT * BTE

NUM_WORKERS = 32
TPW = TOKENS // NUM_WORKERS
CHUNK = 32


def _gating_body(x_ref, wg_ref, pos0_ref, pos1_ref, w0_ref, w1_ref,
                 te_ref, aux_ref):
    x = x_ref[...]
    wg = wg_ref[...]
    logits = lax.dot_general(x, wg, (((1,), (1,)), ((), ())),
                             preferred_element_type=jnp.float32)
    m = jnp.max(logits, axis=-1, keepdims=True)
    ex = jnp.exp(logits - m)
    probs = ex / jnp.sum(ex, axis=-1, keepdims=True)

    eidx = lax.broadcasted_iota(jnp.int32, (TOKENS, NUM_EXPERTS), 1)
    v0 = jnp.max(probs, axis=-1, keepdims=True)
    i0 = jnp.min(jnp.where(probs == v0, eidx, NUM_EXPERTS), axis=-1,
                 keepdims=True)
    probs2 = jnp.where(eidx == i0, -jnp.inf, probs)
    v1 = jnp.max(probs2, axis=-1, keepdims=True)
    i1 = jnp.min(jnp.where(probs2 == v1, eidx, NUM_EXPERTS), axis=-1,
                 keepdims=True)
    wsum = v0 + v1
    w0_ref[...] = jnp.broadcast_to(v0 / wsum, (TOKENS, 16))
    w1_ref[...] = jnp.broadcast_to(v1 / wsum, (TOKENS, 16))

    density = jnp.mean(probs, axis=0)
    proxy = jnp.mean(logits, axis=0)
    aux_ref[...] = jnp.sum(density * proxy).reshape(1, 1) * NUM_EXPERTS

    h0 = (eidx == i0).astype(jnp.int32)
    h1 = (eidx == i1).astype(jnp.int32)
    nb = TOKENS // 128
    hr = (h0 + h1).astype(jnp.float32).reshape(nb, 128, NUM_EXPERTS)
    ri = lax.broadcasted_iota(jnp.int32, (128, 128), 0)
    ci = lax.broadcasted_iota(jnp.int32, (128, 128), 1)
    tri = jnp.broadcast_to((ri >= ci).astype(jnp.float32),
                           (nb, 128, 128))
    cw = lax.dot_general(tri, hr, (((2,), (1,)), ((0,), (0,))),
                         preferred_element_type=jnp.float32)
    s = cw[:, 127, :]
    sx = s
    shift = 1
    while shift < nb:
        sx = sx + jnp.concatenate(
            [jnp.zeros((shift, NUM_EXPERTS), jnp.float32), sx[:-shift]],
            axis=0)
        shift *= 2
    c = (cw + (sx - s)[:, None, :]).reshape(TOKENS, NUM_EXPERTS)
    c = c.astype(jnp.int32)
    counts = sx[nb - 1:nb, :].astype(jnp.int32)
    ntiles = (counts + (BTE - 1)) // BTE
    ct = ntiles
    shift = 1
    while shift < NUM_EXPERTS:
        ct = ct + jnp.concatenate(
            [jnp.zeros((1, shift), jnp.int32), ct[:, :-shift]], axis=1)
        shift *= 2
    cum = jnp.concatenate([jnp.zeros((1, 1), jnp.int32), ct], axis=1)
    row_off = cum[:, :NUM_EXPERTS] * BTE

    pos_of = lambda h: jnp.sum(h * (row_off + c - 1), axis=1)
    pos0_ref[...] = pos_of(h0)
    pos1_ref[...] = pos_of(h1)

    tidx = lax.broadcasted_iota(jnp.int32, (NT, NUM_EXPERTS), 0)
    te = jnp.sum((tidx >= cum[0, 1:][None, :]).astype(jnp.int32), axis=1)
    te_ref[...] = jnp.concatenate(
        [jnp.minimum(te, NUM_EXPERTS - 1), ct[0, -1:]], axis=0)


def _gating(x, wg):
    return pl.pallas_call(
        _gating_body,
        out_shape=(
            jax.ShapeDtypeStruct((TOKENS,), jnp.int32),
            jax.ShapeDtypeStruct((TOKENS,), jnp.int32),
            jax.ShapeDtypeStruct((TOKENS, 16), jnp.float32),
            jax.ShapeDtypeStruct((TOKENS, 16), jnp.float32),
            jax.ShapeDtypeStruct((NT + 1,), jnp.int32),
            jax.ShapeDtypeStruct((1, 1), jnp.float32),
        ),
    )(x, wg)


def _scatter_body(x_hbm, pos0_hbm, pos1_hbm, xs_hbm, x_v, i0_v, i1_v,
                  sem0, sem1):
    wid = lax.axis_index("s") * 2 + lax.axis_index("c")
    base = wid * TPW
    pltpu.sync_copy(x_hbm.at[pl.ds(base, TPW)], x_v)
    pltpu.sync_copy(pos0_hbm.at[pl.ds(base, TPW)], i0_v)
    pltpu.sync_copy(pos1_hbm.at[pl.ds(base, TPW)], i1_v)
    c0 = pltpu.async_copy(x_v, xs_hbm.at[i0_v], sem0)
    c1 = pltpu.async_copy(x_v, xs_hbm.at[i1_v], sem1)
    c0.wait()
    c1.wait()


@functools.cache
def _sc_scatter():
    return pl.kernel(
        _scatter_body,
        out_type=jax.ShapeDtypeStruct((ROWS, DIM), jnp.float32),
        mesh=plsc.VectorSubcoreMesh(core_axis_name="c",
                                    subcore_axis_name="s"),
        scratch_types=[
            pltpu.VMEM((TPW, DIM), jnp.float32),
            pltpu.VMEM((TPW,), jnp.int32),
            pltpu.VMEM((TPW,), jnp.int32),
            pltpu.SemaphoreType.DMA,
            pltpu.SemaphoreType.DMA,
        ],
    )


def _expert_body(te_ref, x_ref, w1_ref, w2_ref, y_ref):
    @pl.when(pl.program_id(0) < te_ref[NT])
    def _():
        xb = x_ref[...]
        h = lax.dot_general(xb, w1_ref[0], (((1,), (1,)), ((), ())),
                            preferred_element_type=jnp.float32)
        h = h * jax.nn.sigmoid(h)
        y_ref[...] = lax.dot_general(h, w2_ref[0], (((1,), (1,)), ((), ())),
                                     preferred_element_type=jnp.float32)


def _expert_gemm(te, xs, wr1, wr2):
    grid_spec = pltpu.PrefetchScalarGridSpec(
        num_scalar_prefetch=1,
        grid=(NT,),
        in_specs=[
            pl.BlockSpec((BTE, DIM), lambda i, te: (i, 0)),
            pl.BlockSpec((1, HIDDEN, DIM), lambda i, te: (te[i], 0, 0)),
            pl.BlockSpec((1, DIM, HIDDEN), lambda i, te: (te[i], 0, 0)),
        ],
        out_specs=pl.BlockSpec((BTE, DIM), lambda i, te: (i, 0)),
    )
    return pl.pallas_call(
        _expert_body,
        grid_spec=grid_spec,
        out_shape=jax.ShapeDtypeStruct((ROWS, DIM), jnp.float32),
    )(te, xs, wr1, wr2)


def _shared_body(x_ref, w1_ref, w2_ref, o_ref):
    xb = x_ref[...]
    acc = jnp.zeros((BT, DIM), jnp.float32)
    for s in range(NUM_SHARED):
        h = lax.dot_general(xb, w1_ref[s], (((1,), (1,)), ((), ())),
                            preferred_element_type=jnp.float32)
        h = h * jax.nn.sigmoid(h)
        acc = acc + lax.dot_general(h, w2_ref[s], (((1,), (1,)), ((), ())),
                                    preferred_element_type=jnp.float32)
    o_ref[...] = acc


def _shared(x, ws1, ws2):
    n = x.shape[0]
    return pl.pallas_call(
        _shared_body,
        grid=(n // BT,),
        in_specs=[
            pl.BlockSpec((BT, DIM), lambda i: (i, 0)),
            pl.BlockSpec((NUM_SHARED, HIDDEN, DIM), lambda i: (0, 0, 0)),
            pl.BlockSpec((NUM_SHARED, DIM, HIDDEN), lambda i: (0, 0, 0)),
        ],
        out_specs=pl.BlockSpec((BT, DIM), lambda i: (i, 0)),
        out_shape=jax.ShapeDtypeStruct((n, DIM), jnp.float32),
    )(x, ws1, ws2)


def _gather_body(y_hbm, pos0_hbm, pos1_hbm, w0_hbm, w1_hbm, o_hbm,
                 y0_v, y1_v, i0_v, i1_v, w0_v, w1_v, sem0, sem1):
    wid = lax.axis_index("s") * 2 + lax.axis_index("c")
    base = wid * TPW
    pltpu.sync_copy(pos0_hbm.at[pl.ds(base, TPW)], i0_v)
    pltpu.sync_copy(pos1_hbm.at[pl.ds(base, TPW)], i1_v)
    c0 = pltpu.async_copy(y_hbm.at[i0_v], y0_v, sem0)
    c1 = pltpu.async_copy(y_hbm.at[i1_v], y1_v, sem1)
    pltpu.sync_copy(w0_hbm.at[pl.ds(base, TPW)], w0_v)
    pltpu.sync_copy(w1_hbm.at[pl.ds(base, TPW)], w1_v)
    c0.wait()
    c1.wait()

    def token(i, _):
        a = w0_v[i, :]
        b = w1_v[i, :]
        for j in range(DIM // 16):
            sl = pl.ds(j * 16, 16)
            y0_v[i, sl] = a * y0_v[i, sl] + b * y1_v[i, sl]
        return 0

    lax.fori_loop(0, TPW, token, 0)
    pltpu.sync_copy(y0_v, o_hbm.at[pl.ds(base, TPW)])


@functools.cache
def _sc_gather():
    return pl.kernel(
        _gather_body,
        out_type=jax.ShapeDtypeStruct((TOKENS, DIM), jnp.float32),
        mesh=plsc.VectorSubcoreMesh(core_axis_name="c",
                                    subcore_axis_name="s"),
        scratch_types=[
            pltpu.VMEM((TPW, DIM), jnp.float32),
            pltpu.VMEM((TPW, DIM), jnp.float32),
            pltpu.VMEM((TPW,), jnp.int32),
            pltpu.VMEM((TPW,), jnp.int32),
            pltpu.VMEM((TPW, 16), jnp.float32),
            pltpu.VMEM((TPW, 16), jnp.float32),
            pltpu.SemaphoreType.DMA,
            pltpu.SemaphoreType.DMA,
        ],
    )


def _combine_body(sh_ref, yr_ref, o_ref):
    o_ref[...] = sh_ref[...] + yr_ref[...]


def _combine(sh, yr):
    return pl.pallas_call(
        _combine_body,
        grid=(TOKENS // BT,),
        in_specs=[
            pl.BlockSpec((BT, DIM), lambda i: (i, 0)),
            pl.BlockSpec((BT, DIM), lambda i: (i, 0)),
        ],
        out_specs=pl.BlockSpec((BT, DIM), lambda i: (i, 0)),
        out_shape=jax.ShapeDtypeStruct((TOKENS, DIM), jnp.float32),
    )(sh, yr)


@jax.jit
def kernel(x, Wg, Ws1, Ws2, Wr1, Wr2):
    shape = x.shape
    xf = x.reshape(TOKENS, DIM)
    pos0, pos1, w0, w1, te, aux = _gating(xf, Wg)
    sh = _shared(xf, Ws1, Ws2)
    xs = _sc_scatter()(xf, pos0, pos1)
    y = _expert_gemm(te, xs, Wr1, Wr2)
    yr = _sc_gather()(y, pos0, pos1, w0, w1)
    out = _combine(sh, yr)
    return out.reshape(shape), aux[0, 0]
